# baseline (speedup 1.0000x reference)
import functools

import jax
import jax.numpy as jnp
from jax import lax
from jax.experimental import pallas as pl
from jax.experimental.pallas import tpu as pltpu
from jax.experimental.pallas import tpu_sc as plsc

_VOCAB = 1000000
_EMB = 64
_B = 4096
_L = 200
_SCALE = 8.0

_NC = 2
_NS = 16
_NW = _NC * _NS

_W = 512
_NBLK = (_VOCAB + 127) // 128
_SCR_ROWS = (_NBLK * 128) // 2
_NBIG = (_NBLK * 128 + _W - 1) // _W
_V_LAST = _NBLK * 128 - _W
_BIG_LO = _NBIG // _NW
_BIG_EXTRA = _NBIG - _BIG_LO * _NW


def _transpose_table(table_t):
    mesh = plsc.VectorSubcoreMesh(core_axis_name="c", subcore_axis_name="s")

    @functools.partial(
        pl.kernel,
        mesh=mesh,
        compiler_params=pltpu.CompilerParams(needs_layout_passes=False),
        out_type=jax.ShapeDtypeStruct((_SCR_ROWS * 128,), jnp.float32),
        scratch_types=[
            pltpu.VMEM((2, _EMB, _W), jnp.float32),
            pltpu.VMEM((_EMB * 128,), jnp.float32),
            pltpu.VMEM((_EMB * 128,), jnp.float32),
            pltpu.VMEM((_EMB * 128,), jnp.float32),
            pltpu.VMEM((_EMB * 128,), jnp.float32),
            pltpu.SemaphoreType.DMA,
            pltpu.SemaphoreType.DMA,
            pltpu.SemaphoreType.DMA,
        ],
    )
    def ka(tab_hbm, scr_hbm, vin, vo0, vo1, vo2, vo3, semi0, semi1, semo):
        vouts = [vo0, vo1, vo2, vo3]
        wid = lax.axis_index("s") * _NC + lax.axis_index("c")
        nblk = jnp.where(wid < _BIG_EXTRA, _BIG_LO + 1, _BIG_LO)
        blk0 = wid * _BIG_LO + jnp.minimum(wid, _BIG_EXTRA)

        lane = lax.iota(jnp.int32, 16)
        evecs = [16 * je + lane for je in range(4)]

        def v0_of(i):
            return pl.multiple_of(
                jnp.minimum((blk0 + i) * _W, _V_LAST), 128
            )

        def vsrc(i):
            return tab_hbm.at[:, pl.ds(v0_of(i), _W)]

        def fire_in(i, buf, sem):
            return pltpu.async_copy(vsrc(i), vin.at[buf], sem)

        def wait_in(i, buf, sem):
            pltpu.make_async_copy(vsrc(i), vin.at[buf], sem).wait()

        def drain_out():
            pltpu.make_async_copy(
                vo0, scr_hbm.at[pl.ds(0, _EMB * 128)], semo
            ).wait()

        def transpose_block(i, buf, first):
            for k4 in range(4):
                @pl.when(jnp.logical_not(first))
                def _():
                    drain_out()

                for g in range(8):
                    v0g = 128 * k4 + 16 * g

                    def k_body(k, carry):
                        vloc = 16 * g + ((lane + k) & 15)
                        vvec = 128 * k4 + vloc
                        vvec64 = vloc * _EMB
                        xs = [
                            plsc.load_gather(vin.at[buf], [evecs[je], vvec])
                            for je in range(4)
                        ]
                        for je in range(4):
                            plsc.store_scatter(
                                vouts[k4], [vvec64 + evecs[je]], xs[je]
                            )
                        return carry

                    lax.fori_loop(0, 16, k_body, 0)
                f0 = pl.multiple_of(
                    (v0_of(i) // 2 + 64 * k4) * 128, _EMB * 128
                )
                pltpu.async_copy(
                    vouts[k4], scr_hbm.at[pl.ds(f0, _EMB * 128)], semo
                )

        fire_in(0, 0, semi0)

        def pair(j, carry):
            i0 = 2 * j

            @pl.when(i0 + 1 < nblk)
            def _():
                fire_in(i0 + 1, 1, semi1)

            @pl.when(i0 < nblk)
            def _():
                wait_in(i0, 0, semi0)
                transpose_block(i0, 0, i0 == 0)

            @pl.when(i0 + 2 < nblk)
            def _():
                fire_in(i0 + 2, 0, semi0)

            @pl.when(i0 + 1 < nblk)
            def _():
                wait_in(i0 + 1, 1, semi1)
                transpose_block(i0 + 1, 1, False)

            return carry

        lax.fori_loop(0, (_BIG_LO + 2) // 2, pair, 0)
        for _ in range(4):
            drain_out()

    return ka(table_t)


def _gather_rows(tokens_t, scr):
    mesh = plsc.VectorSubcoreMesh(core_axis_name="c", subcore_axis_name="s")

    @functools.partial(
        pl.kernel,
        mesh=mesh,
        compiler_params=pltpu.CompilerParams(
            needs_layout_passes=False, use_tc_tiling_on_sc=False
        ),
        out_type=jax.ShapeDtypeStruct((_L, 8, _NW, 8, 128), jnp.float32),
        scratch_types=[
            pltpu.VMEM((_L, 128), jnp.int32),
            pltpu.VMEM((2, 128, _EMB), jnp.float32),
            pltpu.VMEM((2, 8, 8, 128), jnp.float32),
            pltpu.SemaphoreType.DMA,
            pltpu.SemaphoreType.DMA,
            pltpu.SemaphoreType.DMA,
            pltpu.SemaphoreType.DMA,
        ],
    )
    def kb(tok_hbm, scr_hbm, out_hbm, tok_v, rows_v, ob_v,
           semg0, semg1, semw0, semw1):
        wid = lax.axis_index("s") * _NC + lax.axis_index("c")
        b0 = pl.multiple_of(wid * 128, 128)
        lane = lax.iota(jnp.int32, 16)
        evecs = [16 * je + lane for je in range(4)]

        pltpu.sync_copy(tok_hbm.at[:, pl.ds(b0, 128)], tok_v)

        def fire(l, buf, sem):
            return pltpu.async_copy(
                scr_hbm.at[tok_v.at[l]], rows_v.at[buf], sem
            )

        def wait_gather(l, buf, sem):
            pltpu.make_async_copy(
                scr_hbm.at[tok_v.at[l]], rows_v.at[buf], sem
            ).wait()

        def owin(l):
            return out_hbm.at[l, :, wid, :, :]

        def extract(l, buf, semw):
            for g in range(8):
                b0g = 16 * g

                def k_body(k, carry):
                    bvec = b0g + ((lane + k) & 15)
                    xs = [
                        plsc.load_gather(rows_v.at[buf], [bvec, evecs[je]])
                        for je in range(4)
                    ]
                    for je in range(4):
                        evec = evecs[je]
                        plsc.store_scatter(
                            ob_v.at[buf],
                            [evec >> 3, evec & 7, bvec],
                            xs[je] * _SCALE,
                        )
                    return carry

                lax.fori_loop(0, 16, k_body, 0)
            return pltpu.async_copy(ob_v.at[buf], owin(l), semw)

        def drain_write(l, buf, semw):
            pltpu.make_async_copy(ob_v.at[buf], owin(l), semw).wait()

        fire(0, 0, semg0)

        def pair(j, carry):
            l0 = 2 * j
            fire(l0 + 1, 1, semg1)
            wait_gather(l0, 0, semg0)

            @pl.when(l0 >= 2)
            def _():
                drain_write(l0 - 2, 0, semw0)

            extract(l0, 0, semw0)

            @pl.when(l0 + 2 < _L)
            def _():
                fire(l0 + 2, 0, semg0)

            wait_gather(l0 + 1, 1, semg1)

            @pl.when(l0 >= 2)
            def _():
                drain_write(l0 - 1, 1, semw1)

            extract(l0 + 1, 1, semw1)
            return carry

        lax.fori_loop(0, _L // 2, pair, 0)
        drain_write(_L - 2, 0, semw0)
        drain_write(_L - 1, 1, semw1)

    return kb(tokens_t, scr)


def kernel(tokens, table):
    table_t = table.T
    tokens_t = tokens.T.astype(jnp.int32)
    scr = _transpose_table(table_t)
    scr_rows = scr.reshape(2 * _SCR_ROWS, _EMB)
    del scr
    out5 = _gather_rows(tokens_t, scr_rows)
    return out5.transpose(2, 4, 0, 1, 3).reshape(_B, _L, _EMB)
